# baseline (speedup 1.0000x reference)
import functools

import numpy as np
import jax
import jax.numpy as jnp
from jax import lax
from jax.experimental import pallas as pl
from jax.experimental.pallas import tpu as pltpu
from jax.experimental.pallas import tpu_sc as plsc

_MARGIN = 1.0
_MAX_PAIRS = 2000000
_NC, _NS, _L = 2, 16, 16
_NW = _NC * _NS
_B = 4096


_pair_cache = {}


def _rotl(x, d):
    return ((x << np.uint32(d)) | (x >> np.uint32(32 - d))).astype(np.uint32)


def _threefry2x32(keypair, x0, x1):
    x0 = np.asarray(x0, np.uint32).copy()
    x1 = np.asarray(x1, np.uint32).copy()
    ks0 = np.uint32(keypair[0])
    ks1 = np.uint32(keypair[1])
    ks2 = np.uint32(ks0 ^ ks1 ^ np.uint32(0x1BD11BDA))
    rot0 = (13, 15, 26, 6)
    rot1 = (17, 29, 16, 24)

    def rounds(x0, x1, rots):
        for r in rots:
            x0 = (x0 + x1).astype(np.uint32)
            x1 = _rotl(x1, r)
            x1 = x1 ^ x0
        return x0, x1

    x0 = (x0 + ks0).astype(np.uint32)
    x1 = (x1 + ks1).astype(np.uint32)
    x0, x1 = rounds(x0, x1, rot0)
    x0 = (x0 + ks1).astype(np.uint32)
    x1 = (x1 + ks2 + np.uint32(1)).astype(np.uint32)
    x0, x1 = rounds(x0, x1, rot1)
    x0 = (x0 + ks2).astype(np.uint32)
    x1 = (x1 + ks0 + np.uint32(2)).astype(np.uint32)
    x0, x1 = rounds(x0, x1, rot0)
    x0 = (x0 + ks0).astype(np.uint32)
    x1 = (x1 + ks1 + np.uint32(3)).astype(np.uint32)
    x0, x1 = rounds(x0, x1, rot1)
    x0 = (x0 + ks1).astype(np.uint32)
    x1 = (x1 + ks2 + np.uint32(4)).astype(np.uint32)
    x0, x1 = rounds(x0, x1, rot0)
    x0 = (x0 + ks2).astype(np.uint32)
    x1 = (x1 + ks0 + np.uint32(5)).astype(np.uint32)
    return x0, x1


def _np_split(keypair, num=2):
    counts = np.arange(num, dtype=np.uint64)
    b1, b2 = _threefry2x32(keypair, (counts >> np.uint64(32)).astype(np.uint32),
                           (counts & np.uint64(0xFFFFFFFF)).astype(np.uint32))
    return np.stack([b1, b2], axis=1)


def _np_random_bits(keypair, size):
    counts = np.arange(size, dtype=np.uint64)
    b1, b2 = _threefry2x32(keypair, (counts >> np.uint64(32)).astype(np.uint32),
                           (counts & np.uint64(0xFFFFFFFF)).astype(np.uint32))
    return b1 ^ b2


def _np_randint(keypair, size, minval, maxval):
    khi, klo = _np_split(keypair, 2)
    higher = _np_random_bits(khi, size)
    lower = _np_random_bits(klo, size)
    span = np.uint32(maxval - minval)
    multiplier = np.uint32((int(np.uint32(65536) % span) ** 2) & 0xFFFFFFFF) % span
    with np.errstate(over="ignore"):
        offset = ((higher % span) * multiplier + (lower % span)) % span
    return (np.int32(minval) + offset.astype(np.int32)).astype(np.int32)


def _pair_layout(n):
    if n in _pair_cache:
        return _pair_cache[n]
    n_pairs = min(_MAX_PAIRS, n * (n - 1) // 2)
    root = np.array([0, 42], np.uint32)
    ki, kj = _np_split(root, 2)
    idx_i = _np_randint(ki, n_pairs, 0, n)
    idx_j = _np_randint(kj, n_pairs, 0, n)
    keep = idx_i != idx_j
    idx_i, idx_j = idx_i[keep], idx_j[keep]
    m = idx_i.shape[0]
    nch = -(-(-(-m // _NW)) // _B)
    c_tile = nch * _B
    total = c_tile * _NW
    ii = np.zeros((total,), np.int32)
    jj = np.zeros((total,), np.int32)
    ii[:m] = idx_i
    jj[:m] = idx_j
    idx = np.stack([ii.reshape(_NW, nch, _B), jj.reshape(_NW, nch, _B)], axis=2)
    out = (jnp.asarray(idx), nch, c_tile)
    _pair_cache[n] = out
    return out


def _final_kernel(lp, cp, o):
    s = jnp.sum(lp[...])
    c = jnp.sum(cp[...])
    o[0] = s / jnp.maximum(c, 1.0)


def _make_sc_loss(n, nch, c_tile, npt):
    mesh = plsc.VectorSubcoreMesh(core_axis_name="c", subcore_axis_name="s")
    nprime = npt * _NS

    @functools.partial(
        pl.kernel,
        out_type=[
            jax.ShapeDtypeStruct((_NW, _L), jnp.float32),
            jax.ShapeDtypeStruct((_NW, _L), jnp.float32),
            jax.ShapeDtypeStruct((_NW, c_tile // 2), jnp.float32),
            jax.ShapeDtypeStruct((n,), jnp.float32),
        ],
        mesh=mesh,
        compiler_params=pltpu.CompilerParams(
            needs_layout_passes=False, use_tc_tiling_on_sc=False),
        scratch_types=[
            pltpu.VMEM((n,), jnp.float32),
            pltpu.VMEM((2, 2, _B), jnp.int32),
            pltpu.VMEM((2 * _B,), jnp.float32),
            pltpu.SemaphoreType.DMA((2,)),
            pltpu.SemaphoreType.DMA((2,)),
            pltpu.SemaphoreType.DMA,
        ],
    )
    def sc_loss(pred_hbm, cic_hbm, idx_hbm, loss_out, cnt_out, stage, ctab,
                table, idxb, pdb, isems, psems, tsem):
        cid = lax.axis_index("c")
        sid = lax.axis_index("s")
        wid = sid * _NC + cid
        nvec = _B // _L

        in_d = [None] * nch
        out_d = [None] * nch
        in_d[0] = pltpu.async_copy(idx_hbm.at[wid, 0], idxb.at[0], isems.at[0])
        tbl_d = pltpu.async_copy(pred_hbm, table, tsem)
        sub = 1024
        p0out = 4 * sub

        def p0_block(so, cnt):
            base = (sid * npt + so) * 4
            pltpu.sync_copy(cic_hbm.at[pl.ds(base, cnt * 4)],
                            pdb.at[pl.ds(0, cnt * 4)])

            nv0 = cnt // _L

            @plsc.parallel_loop(0, nv0, step=1,
                                unroll=(4 if nv0 % 4 == 0 else 1))
            def p0(v):
                i0 = lax.iota(jnp.int32, _L) * 4 + v * 64
                c0 = plsc.load_gather(pdb, [i0])
                c1 = plsc.load_gather(pdb, [i0 + 1])
                c2 = plsc.load_gather(pdb, [i0 + 2])
                c3 = plsc.load_gather(pdb, [i0 + 3])
                t = ((1.0 - 0.25 * jnp.clip(c0, 0.0, 1.0))
                     * (1.0 - 0.25 * jnp.clip(c1, 0.0, 1.0))
                     * (1.0 - 0.25 * jnp.clip(c2, 0.0, 1.0))
                     * (1.0 - 0.25 * jnp.clip(c3, 0.0, 1.0)))
                pdb[pl.ds(p0out + v * _L, _L)] = 1.0 - t

            pltpu.sync_copy(pdb.at[pl.ds(p0out, cnt)],
                            ctab.at[pl.ds(sid * npt + so, cnt)])

        for so in range(0, npt, sub):
            cnt = min(sub, npt - so)
            cnt_last = min(cnt, n - (_NS - 1) * npt - so)
            if cnt_last == cnt:
                p0_block(so, cnt)
            else:
                @pl.when(sid < _NS - 1)
                def _():
                    p0_block(so, cnt)
                if cnt_last > 0:
                    @pl.when(sid == _NS - 1)
                    def _():
                        p0_block(so, cnt_last)
        plsc.subcore_barrier()

        tbl_d.wait()
        for ch in range(nch):
            cur = ch % 2
            if ch + 1 < nch:
                in_d[ch + 1] = pltpu.async_copy(
                    idx_hbm.at[wid, ch + 1], idxb.at[1 - cur], isems.at[1 - cur])
            in_d[ch].wait()
            if ch >= 2:
                out_d[ch - 2].wait()

            @plsc.parallel_loop(0, nvec // 2, step=1, unroll=4)
            def p1(w):
                off = pl.multiple_of(w * 2 * _L, _L)
                ii0 = idxb[cur, 0, pl.ds(off, _L)]
                jj0 = idxb[cur, 1, pl.ds(off, _L)]
                ii1 = idxb[cur, 0, pl.ds(off + _L, _L)]
                jj1 = idxb[cur, 1, pl.ds(off + _L, _L)]
                pd0 = plsc.load_gather(table, [ii0]) - plsc.load_gather(table, [jj0])
                pd1 = plsc.load_gather(table, [ii1]) - plsc.load_gather(table, [jj1])
                packed = plsc.pack(pd0, pd1, format=plsc.PackFormat.INTERLEAVED)
                pdb[pl.ds(cur * _B + w * _L, _L)] = plsc.bitcast(packed, jnp.float32)
            out_d[ch] = pltpu.async_copy(
                pdb.at[pl.ds(cur * _B, _B // 2)],
                stage.at[wid, pl.ds(ch * (_B // 2), _B // 2)],
                psems.at[cur])
        out_d[nch - 2].wait()
        out_d[nch - 1].wait()

        in_d = [None] * nch
        pd_d = [None] * nch
        in_d[0] = pltpu.async_copy(idx_hbm.at[wid, 0], idxb.at[0], isems.at[0])
        pd_d[0] = pltpu.async_copy(
            stage.at[wid, pl.ds(0, _B // 2)], pdb.at[pl.ds(0, _B // 2)],
            psems.at[0])
        pltpu.sync_copy(ctab, table)
        acc = (jnp.zeros((_L,), jnp.float32), jnp.zeros((_L,), jnp.float32))
        for ch in range(nch):
            cur = ch % 2
            if ch + 1 < nch:
                in_d[ch + 1] = pltpu.async_copy(
                    idx_hbm.at[wid, ch + 1], idxb.at[1 - cur], isems.at[1 - cur])
                pd_d[ch + 1] = pltpu.async_copy(
                    stage.at[wid, pl.ds((ch + 1) * (_B // 2), _B // 2)],
                    pdb.at[pl.ds((1 - cur) * _B, _B // 2)], psems.at[1 - cur])
            in_d[ch].wait()
            pd_d[ch].wait()

            def p2(w, carry):
                al, ac = carry
                off = pl.multiple_of(w * 2 * _L, _L)
                packed = plsc.bitcast(pdb[pl.ds(cur * _B + w * _L, _L)],
                                      jnp.bfloat16)
                pd0, pd1 = plsc.unpack(packed, format=plsc.PackFormat.INTERLEAVED)
                for half, pdv in ((0, pd0), (1, pd1)):
                    o = off + half * _L
                    ii = idxb[cur, 0, pl.ds(o, _L)]
                    jj = idxb[cur, 1, pl.ds(o, _L)]
                    ci = plsc.load_gather(table, [ii])
                    cj = plsc.load_gather(table, [jj])
                    cd = ci - cj
                    sgn = jnp.sign(cd)
                    elem = jnp.maximum(_MARGIN - sgn * pdv.astype(jnp.float32),
                                       0.0)
                    mf = jnp.where(jnp.abs(cd) > 0.1, 1.0, 0.0)
                    al = al + elem * mf
                    ac = ac + mf
                return (al, ac)

            acc = plsc.parallel_loop(0, nvec // 2, step=1, unroll=2, carry=acc)(p2)

        pdb[pl.ds(0, _L)] = acc[0]
        pdb[pl.ds(_L, _L)] = acc[1]
        pltpu.sync_copy(pdb.at[pl.ds(0, _L)], loss_out.at[wid])
        pltpu.sync_copy(pdb.at[pl.ds(_L, _L)], cnt_out.at[wid])

    return sc_loss


def kernel(pred_scores, cic_scores):
    pred = pred_scores.reshape(-1).astype(jnp.float32)
    n = pred.shape[0]
    idx, nch, c_tile = _pair_layout(n)

    npt = -(-n // (_NS * _L)) * _L
    cic_flat = cic_scores.astype(jnp.float32).reshape(-1)

    sc_loss = _make_sc_loss(n, nch, c_tile, npt)
    loss_part, cnt_part, _, _ = sc_loss(pred, cic_flat, idx)

    out = pl.pallas_call(
        _final_kernel,
        out_shape=jax.ShapeDtypeStruct((1,), jnp.float32),
        out_specs=pl.BlockSpec(memory_space=pltpu.SMEM),
    )(loss_part, cnt_part)
    return out[0]


# speedup vs baseline: 1.2216x; 1.2216x over previous
import functools

import numpy as np
import jax
import jax.numpy as jnp
from jax import lax
from jax.experimental import pallas as pl
from jax.experimental.pallas import tpu as pltpu
from jax.experimental.pallas import tpu_sc as plsc

_MARGIN = 1.0
_MAX_PAIRS = 2000000
_NC, _NS, _L = 2, 16, 16
_NW = _NC * _NS
_B = 4096


_pair_cache = {}


def _rotl(x, d):
    return ((x << np.uint32(d)) | (x >> np.uint32(32 - d))).astype(np.uint32)


def _threefry2x32(keypair, x0, x1):
    x0 = np.asarray(x0, np.uint32).copy()
    x1 = np.asarray(x1, np.uint32).copy()
    ks0 = np.uint32(keypair[0])
    ks1 = np.uint32(keypair[1])
    ks2 = np.uint32(ks0 ^ ks1 ^ np.uint32(0x1BD11BDA))
    rot0 = (13, 15, 26, 6)
    rot1 = (17, 29, 16, 24)

    def rounds(x0, x1, rots):
        for r in rots:
            x0 = (x0 + x1).astype(np.uint32)
            x1 = _rotl(x1, r)
            x1 = x1 ^ x0
        return x0, x1

    x0 = (x0 + ks0).astype(np.uint32)
    x1 = (x1 + ks1).astype(np.uint32)
    x0, x1 = rounds(x0, x1, rot0)
    x0 = (x0 + ks1).astype(np.uint32)
    x1 = (x1 + ks2 + np.uint32(1)).astype(np.uint32)
    x0, x1 = rounds(x0, x1, rot1)
    x0 = (x0 + ks2).astype(np.uint32)
    x1 = (x1 + ks0 + np.uint32(2)).astype(np.uint32)
    x0, x1 = rounds(x0, x1, rot0)
    x0 = (x0 + ks0).astype(np.uint32)
    x1 = (x1 + ks1 + np.uint32(3)).astype(np.uint32)
    x0, x1 = rounds(x0, x1, rot1)
    x0 = (x0 + ks1).astype(np.uint32)
    x1 = (x1 + ks2 + np.uint32(4)).astype(np.uint32)
    x0, x1 = rounds(x0, x1, rot0)
    x0 = (x0 + ks2).astype(np.uint32)
    x1 = (x1 + ks0 + np.uint32(5)).astype(np.uint32)
    return x0, x1


def _np_split(keypair, num=2):
    counts = np.arange(num, dtype=np.uint64)
    b1, b2 = _threefry2x32(keypair, (counts >> np.uint64(32)).astype(np.uint32),
                           (counts & np.uint64(0xFFFFFFFF)).astype(np.uint32))
    return np.stack([b1, b2], axis=1)


def _np_random_bits(keypair, size):
    counts = np.arange(size, dtype=np.uint64)
    b1, b2 = _threefry2x32(keypair, (counts >> np.uint64(32)).astype(np.uint32),
                           (counts & np.uint64(0xFFFFFFFF)).astype(np.uint32))
    return b1 ^ b2


def _np_randint(keypair, size, minval, maxval):
    khi, klo = _np_split(keypair, 2)
    higher = _np_random_bits(khi, size)
    lower = _np_random_bits(klo, size)
    span = np.uint32(maxval - minval)
    multiplier = np.uint32((int(np.uint32(65536) % span) ** 2) & 0xFFFFFFFF) % span
    with np.errstate(over="ignore"):
        offset = ((higher % span) * multiplier + (lower % span)) % span
    return (np.int32(minval) + offset.astype(np.int32)).astype(np.int32)


def _pair_layout(n):
    if n in _pair_cache:
        return _pair_cache[n]
    n_pairs = min(_MAX_PAIRS, n * (n - 1) // 2)
    root = np.array([0, 42], np.uint32)
    ki, kj = _np_split(root, 2)
    idx_i = _np_randint(ki, n_pairs, 0, n)
    idx_j = _np_randint(kj, n_pairs, 0, n)
    keep = idx_i != idx_j
    idx_i, idx_j = idx_i[keep], idx_j[keep]
    m = idx_i.shape[0]
    nch = -(-(-(-m // _NW)) // _B)
    c_tile = nch * _B
    total = c_tile * _NW
    ii = np.zeros((total,), np.int32)
    jj = np.zeros((total,), np.int32)
    ii[:m] = idx_i
    jj[:m] = idx_j
    idx = np.stack([ii.reshape(_NW, nch, _B), jj.reshape(_NW, nch, _B)], axis=2)
    out = (jnp.asarray(idx), nch, c_tile)
    _pair_cache[n] = out
    return out


def _final_kernel(lp, cp, o):
    s = jnp.sum(lp[...])
    c = jnp.sum(cp[...])
    o[0] = s / jnp.maximum(c, 1.0)


def _make_sc_loss(n, nch, c_tile, npt):
    mesh = plsc.VectorSubcoreMesh(core_axis_name="c", subcore_axis_name="s")

    @functools.partial(
        pl.kernel,
        out_type=[
            jax.ShapeDtypeStruct((_NW, _L), jnp.float32),
            jax.ShapeDtypeStruct((_NW, _L), jnp.float32),
            jax.ShapeDtypeStruct((n,), jnp.float32),
        ],
        mesh=mesh,
        compiler_params=pltpu.CompilerParams(
            needs_layout_passes=False, use_tc_tiling_on_sc=False),
        scratch_types=[
            pltpu.VMEM((n,), jnp.float32),
            pltpu.VMEM((2, 2, _B), jnp.int32),
            pltpu.VMEM((2 * _B,), jnp.float32),
            pltpu.SemaphoreType.DMA((2,)),
            pltpu.SemaphoreType.DMA,
        ],
    )
    def sc_loss(pred_hbm, cic_hbm, idx_hbm, loss_out, cnt_out, ctab,
                table, idxb, pdb, isems, tsem):
        cid = lax.axis_index("c")
        sid = lax.axis_index("s")
        wid = sid * _NC + cid
        nvec = _B // _L

        in_d = [None] * nch
        in_d[0] = pltpu.async_copy(idx_hbm.at[wid, 0], idxb.at[0], isems.at[0])
        pin = npt * 4
        pout = pin + npt

        def p0_block(cnt):
            d1 = pltpu.async_copy(cic_hbm.at[pl.ds(sid * npt * 4, cnt * 4)],
                                  table.at[pl.ds(0, cnt * 4)], tsem)
            d2 = pltpu.async_copy(pred_hbm.at[pl.ds(sid * npt, cnt)],
                                  table.at[pl.ds(pin, cnt)], tsem)
            d1.wait()
            d2.wait()

            nv0 = cnt // _L
            nv_main = nv0 - nv0 % 4

            def p0(v):
                i0 = lax.iota(jnp.int32, _L) * 4 + v * 64
                c0 = plsc.load_gather(table, [i0])
                c1 = plsc.load_gather(table, [i0 + 1])
                c2 = plsc.load_gather(table, [i0 + 2])
                c3 = plsc.load_gather(table, [i0 + 3])
                t = ((1.0 - 0.25 * jnp.clip(c0, 0.0, 1.0))
                     * (1.0 - 0.25 * jnp.clip(c1, 0.0, 1.0))
                     * (1.0 - 0.25 * jnp.clip(c2, 0.0, 1.0))
                     * (1.0 - 0.25 * jnp.clip(c3, 0.0, 1.0)))
                pr = table[pl.ds(pin + v * _L, _L)]
                packed = plsc.pack(pr, 1.0 - t,
                                   format=plsc.PackFormat.INTERLEAVED)
                table[pl.ds(pout + v * _L, _L)] = plsc.bitcast(packed,
                                                               jnp.float32)

            plsc.parallel_loop(0, nv_main, step=1, unroll=4)(p0)
            if nv0 % 4:
                plsc.parallel_loop(nv_main, nv0, step=1, unroll=1)(p0)
            pltpu.sync_copy(table.at[pl.ds(pout, cnt)],
                            ctab.at[pl.ds(sid * npt, cnt)])

        plast = n - (_NS - 1) * npt
        if plast == npt:
            p0_block(npt)
        else:
            @pl.when(sid < _NS - 1)
            def _():
                p0_block(npt)

            @pl.when(sid == _NS - 1)
            def _():
                p0_block(plast)
        plsc.subcore_barrier()

        pltpu.async_copy(ctab, table, tsem).wait()
        acc = (jnp.zeros((_L,), jnp.float32), jnp.zeros((_L,), jnp.float32))
        for ch in range(nch):
            cur = ch % 2
            if ch + 1 < nch:
                in_d[ch + 1] = pltpu.async_copy(
                    idx_hbm.at[wid, ch + 1], idxb.at[1 - cur], isems.at[1 - cur])
            in_d[ch].wait()

            def p2(v, carry):
                al, ac = carry
                off = pl.multiple_of(v * _L, _L)
                ii = idxb[cur, 0, pl.ds(off, _L)]
                jj = idxb[cur, 1, pl.ds(off, _L)]
                gi = plsc.load_gather(table, [ii])
                gj = plsc.load_gather(table, [jj])
                pi, ci = plsc.unpack(plsc.bitcast(gi, jnp.bfloat16),
                                     format=plsc.PackFormat.INTERLEAVED)
                pj, cj = plsc.unpack(plsc.bitcast(gj, jnp.bfloat16),
                                     format=plsc.PackFormat.INTERLEAVED)
                pd = pi - pj
                cd = ci - cj
                sbit = plsc.bitcast(cd, jnp.int32) & jnp.int32(-2147483648)
                pdx = plsc.bitcast(plsc.bitcast(pd, jnp.int32) ^ sbit,
                                   jnp.float32)
                elem = jnp.maximum(_MARGIN - pdx, 0.0)
                mf = jnp.where(jnp.abs(cd) > 0.1, 1.0, 0.0)
                return (al + elem * mf, ac + mf)

            acc = plsc.parallel_loop(0, nvec, step=1, unroll=4, carry=acc)(p2)

        pdb[pl.ds(0, _L)] = acc[0]
        pdb[pl.ds(_L, _L)] = acc[1]
        pltpu.sync_copy(pdb.at[pl.ds(0, _L)], loss_out.at[wid])
        pltpu.sync_copy(pdb.at[pl.ds(_L, _L)], cnt_out.at[wid])

    return sc_loss


def kernel(pred_scores, cic_scores):
    pred = pred_scores.reshape(-1).astype(jnp.float32)
    n = pred.shape[0]
    idx, nch, c_tile = _pair_layout(n)

    npt = -(-n // (_NS * _L)) * _L
    cic_flat = cic_scores.astype(jnp.float32).reshape(-1)

    sc_loss = _make_sc_loss(n, nch, c_tile, npt)
    loss_part, cnt_part, _ = sc_loss(pred, cic_flat, idx)

    out = pl.pallas_call(
        _final_kernel,
        out_shape=jax.ShapeDtypeStruct((1,), jnp.float32),
        out_specs=pl.BlockSpec(memory_space=pltpu.SMEM),
    )(loss_part, cnt_part)
    return out[0]


# speedup vs baseline: 2.2720x; 1.8598x over previous
import functools

import numpy as np
import jax
import jax.numpy as jnp
from jax import lax
from jax.experimental import pallas as pl
from jax.experimental.pallas import tpu as pltpu
from jax.experimental.pallas import tpu_sc as plsc

_MARGIN = 1.0
_MAX_PAIRS = 2000000
_NC, _NS, _L = 2, 16, 16
_NW = _NC * _NS
_B = 4096


_pair_cache = {}


def _rotl(x, d):
    return ((x << np.uint32(d)) | (x >> np.uint32(32 - d))).astype(np.uint32)


def _threefry2x32(keypair, x0, x1):
    x0 = np.asarray(x0, np.uint32).copy()
    x1 = np.asarray(x1, np.uint32).copy()
    ks0 = np.uint32(keypair[0])
    ks1 = np.uint32(keypair[1])
    ks2 = np.uint32(ks0 ^ ks1 ^ np.uint32(0x1BD11BDA))
    rot0 = (13, 15, 26, 6)
    rot1 = (17, 29, 16, 24)

    def rounds(x0, x1, rots):
        for r in rots:
            x0 = (x0 + x1).astype(np.uint32)
            x1 = _rotl(x1, r)
            x1 = x1 ^ x0
        return x0, x1

    x0 = (x0 + ks0).astype(np.uint32)
    x1 = (x1 + ks1).astype(np.uint32)
    x0, x1 = rounds(x0, x1, rot0)
    x0 = (x0 + ks1).astype(np.uint32)
    x1 = (x1 + ks2 + np.uint32(1)).astype(np.uint32)
    x0, x1 = rounds(x0, x1, rot1)
    x0 = (x0 + ks2).astype(np.uint32)
    x1 = (x1 + ks0 + np.uint32(2)).astype(np.uint32)
    x0, x1 = rounds(x0, x1, rot0)
    x0 = (x0 + ks0).astype(np.uint32)
    x1 = (x1 + ks1 + np.uint32(3)).astype(np.uint32)
    x0, x1 = rounds(x0, x1, rot1)
    x0 = (x0 + ks1).astype(np.uint32)
    x1 = (x1 + ks2 + np.uint32(4)).astype(np.uint32)
    x0, x1 = rounds(x0, x1, rot0)
    x0 = (x0 + ks2).astype(np.uint32)
    x1 = (x1 + ks0 + np.uint32(5)).astype(np.uint32)
    return x0, x1


def _np_split(keypair, num=2):
    counts = np.arange(num, dtype=np.uint64)
    b1, b2 = _threefry2x32(keypair, (counts >> np.uint64(32)).astype(np.uint32),
                           (counts & np.uint64(0xFFFFFFFF)).astype(np.uint32))
    return np.stack([b1, b2], axis=1)


def _np_random_bits(keypair, size):
    counts = np.arange(size, dtype=np.uint64)
    b1, b2 = _threefry2x32(keypair, (counts >> np.uint64(32)).astype(np.uint32),
                           (counts & np.uint64(0xFFFFFFFF)).astype(np.uint32))
    return b1 ^ b2


def _np_randint(keypair, size, minval, maxval):
    khi, klo = _np_split(keypair, 2)
    higher = _np_random_bits(khi, size)
    lower = _np_random_bits(klo, size)
    span = np.uint32(maxval - minval)
    multiplier = np.uint32((int(np.uint32(65536) % span) ** 2) & 0xFFFFFFFF) % span
    with np.errstate(over="ignore"):
        offset = ((higher % span) * multiplier + (lower % span)) % span
    return (np.int32(minval) + offset.astype(np.int32)).astype(np.int32)


def _pair_layout(n):
    if n in _pair_cache:
        return _pair_cache[n]
    n_pairs = min(_MAX_PAIRS, n * (n - 1) // 2)
    root = np.array([0, 42], np.uint32)
    ki, kj = _np_split(root, 2)
    idx_i = _np_randint(ki, n_pairs, 0, n)
    idx_j = _np_randint(kj, n_pairs, 0, n)
    keep = idx_i != idx_j
    idx_i, idx_j = idx_i[keep], idx_j[keep]
    m = idx_i.shape[0]
    nch = -(-(-(-m // _NW)) // _B)
    c_tile = nch * _B
    total = c_tile * _NW
    ii = np.zeros((total,), np.int32)
    jj = np.zeros((total,), np.int32)
    ii[:m] = idx_i
    jj[:m] = idx_j
    idx = np.stack([ii.reshape(_NW, nch, _B), jj.reshape(_NW, nch, _B)], axis=2)
    out = (jnp.asarray(idx), nch, c_tile)
    _pair_cache[n] = out
    return out


def _final_kernel(lp, cp, o):
    s = jnp.sum(lp[...])
    c = jnp.sum(cp[...])
    o[0] = s / jnp.maximum(c, 1.0)


def _pack_kernel(c0, c1, c2, c3, pr, o):
    t0 = 1.0 - 0.25 * jnp.clip(c0[...], 0.0, 1.0)
    t1 = 1.0 - 0.25 * jnp.clip(c1[...], 0.0, 1.0)
    t2 = 1.0 - 0.25 * jnp.clip(c2[...], 0.0, 1.0)
    t3 = 1.0 - 0.25 * jnp.clip(c3[...], 0.0, 1.0)
    ct = 1.0 - t0 * t1 * t2 * t3
    pb = lax.bitcast_convert_type(pr[...].astype(jnp.bfloat16), jnp.uint16)
    cb = lax.bitcast_convert_type(ct.astype(jnp.bfloat16), jnp.uint16)
    word = (cb.astype(jnp.uint32) << 16) | pb.astype(jnp.uint32)
    o[...] = lax.bitcast_convert_type(word, jnp.float32)


def _make_sc_loss(n, nch, c_tile):
    mesh = plsc.VectorSubcoreMesh(core_axis_name="c", subcore_axis_name="s")

    @functools.partial(
        pl.kernel,
        out_type=[
            jax.ShapeDtypeStruct((_NW, _L), jnp.float32),
            jax.ShapeDtypeStruct((_NW, _L), jnp.float32),
        ],
        mesh=mesh,
        compiler_params=pltpu.CompilerParams(
            needs_layout_passes=False, use_tc_tiling_on_sc=False),
        scratch_types=[
            pltpu.VMEM((n,), jnp.float32),
            pltpu.VMEM((2, 2, _B), jnp.int32),
            pltpu.VMEM((2 * _L,), jnp.float32),
            pltpu.SemaphoreType.DMA((2,)),
            pltpu.SemaphoreType.DMA,
        ],
    )
    def sc_loss(tab_hbm, idx_hbm, loss_out, cnt_out, table, idxb, pout, isems,
                tsem):
        cid = lax.axis_index("c")
        sid = lax.axis_index("s")
        wid = sid * _NC + cid
        nvec = _B // _L

        in_d = [None] * nch
        in_d[0] = pltpu.async_copy(idx_hbm.at[wid, 0], idxb.at[0], isems.at[0])
        pltpu.async_copy(tab_hbm.at[pl.ds(0, n)], table, tsem).wait()
        acc = (jnp.zeros((_L,), jnp.float32), jnp.zeros((_L,), jnp.float32))
        for ch in range(nch):
            cur = ch % 2
            if ch + 1 < nch:
                in_d[ch + 1] = pltpu.async_copy(
                    idx_hbm.at[wid, ch + 1], idxb.at[1 - cur], isems.at[1 - cur])
            in_d[ch].wait()

            def p2(v, carry):
                al, ac = carry
                off = pl.multiple_of(v * _L, _L)
                ii = idxb[cur, 0, pl.ds(off, _L)]
                jj = idxb[cur, 1, pl.ds(off, _L)]
                gi = plsc.load_gather(table, [ii])
                gj = plsc.load_gather(table, [jj])
                pi, ci = plsc.unpack(plsc.bitcast(gi, jnp.bfloat16),
                                     format=plsc.PackFormat.INTERLEAVED)
                pj, cj = plsc.unpack(plsc.bitcast(gj, jnp.bfloat16),
                                     format=plsc.PackFormat.INTERLEAVED)
                pd = pi - pj
                cd = ci - cj
                sbit = plsc.bitcast(cd, jnp.int32) & jnp.int32(-2147483648)
                pdx = plsc.bitcast(plsc.bitcast(pd, jnp.int32) ^ sbit,
                                   jnp.float32)
                elem = jnp.maximum(_MARGIN - pdx, 0.0)
                mf = jnp.where(jnp.abs(cd) > 0.1, 1.0, 0.0)
                return (al + elem * mf, ac + mf)

            acc = plsc.parallel_loop(0, nvec, step=1, unroll=4, carry=acc)(p2)

        pout[pl.ds(0, _L)] = acc[0]
        pout[pl.ds(_L, _L)] = acc[1]
        pltpu.sync_copy(pout.at[pl.ds(0, _L)], loss_out.at[wid])
        pltpu.sync_copy(pout.at[pl.ds(_L, _L)], cnt_out.at[wid])

    return sc_loss


def kernel(pred_scores, cic_scores):
    pred = pred_scores.reshape(-1).astype(jnp.float32)
    n = pred.shape[0]
    idx, nch, c_tile = _pair_layout(n)

    npad = -(-n // 128) * 128
    rows = npad // 128
    cic_t = jnp.pad(cic_scores.astype(jnp.float32), ((0, npad - n), (0, 0))).T
    cols = cic_t.reshape(4, rows, 128)
    pred_rows = jnp.pad(pred, (0, npad - n)).reshape(rows, 128)
    packed = pl.pallas_call(
        _pack_kernel,
        out_shape=jax.ShapeDtypeStruct((rows, 128), jnp.float32),
    )(cols[0], cols[1], cols[2], cols[3], pred_rows).reshape(npad)

    sc_loss = _make_sc_loss(n, nch, c_tile)
    loss_part, cnt_part = sc_loss(packed, idx)

    out = pl.pallas_call(
        _final_kernel,
        out_shape=jax.ShapeDtypeStruct((1,), jnp.float32),
        out_specs=pl.BlockSpec(memory_space=pltpu.SMEM),
    )(loss_part, cnt_part)
    return out[0]
